# baseline (speedup 1.0000x reference)
import functools

import jax
import jax.numpy as jnp
from jax import lax
from jax.experimental import pallas as pl
from jax.experimental.pallas import tpu as pltpu
from jax.experimental.pallas import tpu_sc as plsc

N = 10000
E = 320000
D_IN = 128
D_HID = 256
D_OUT = 128

NC = 2
NS = 16
NW = NC * NS
LANE = 128
HW = 64
N_PAD = 10240
ROWS_PER_TILE = N_PAD // NS
NCH = 80
E_PAD = NW * NCH * LANE
NBUF = 2


def _make_sc_agg(with_counts, W=HW, lane=LANE, nch=NCH, stage_table=True,
                 async_scatter=False):
    npass = D_IN // W
    mesh = plsc.VectorSubcoreMesh(core_axis_name="c", subcore_axis_name="s")
    out_type = jax.ShapeDtypeStruct((NC, npass, N_PAD, W), jnp.float32)
    if with_counts:
        out_type = (out_type,
                    jax.ShapeDtypeStruct((NW, N_PAD), jnp.float32))
    nbuf = 3 if async_scatter else NBUF
    scratch = [
        pltpu.VMEM((nch, lane), jnp.int32),
        pltpu.VMEM((nch, lane), jnp.int32),
        [pltpu.VMEM((lane, W), jnp.float32) for _ in range(nbuf)],
        pltpu.VMEM_SHARED((N_PAD, W), jnp.float32),
        [pltpu.SemaphoreType.DMA for _ in range(nbuf)],
        [pltpu.SemaphoreType.DMA for _ in range(nbuf)],
    ]
    if stage_table:
        scratch.append(pltpu.VMEM_SHARED((N_PAD, W), jnp.float32))
    if with_counts:
        scratch.append(pltpu.VMEM((N_PAD,), jnp.float32))

    @functools.partial(
        pl.kernel, out_type=out_type, mesh=mesh, scratch_types=scratch,
        compiler_params=pltpu.CompilerParams(needs_layout_passes=False,
                                             use_tc_tiling_on_sc=False))
    def agg(*args):
        tables = args[:npass]
        src, dst, zeros = args[npass:npass + 3]
        rest = args[npass + 3:]
        table_sh = None
        if with_counts and stage_table:
            out, cnt_out, src_v, dst_v, bufs, acc, gsem, ssem, table_sh, cnt_v = rest
        elif with_counts:
            out, cnt_out, src_v, dst_v, bufs, acc, gsem, ssem, cnt_v = rest
        elif stage_table:
            out, src_v, dst_v, bufs, acc, gsem, ssem, table_sh = rest
            cnt_out = cnt_v = None
        else:
            out, src_v, dst_v, bufs, acc, gsem, ssem = rest
            cnt_out = cnt_v = None
        c = lax.axis_index("c")
        s = lax.axis_index("s")
        wid = s * NC + c
        row0 = s * ROWS_PER_TILE
        pltpu.sync_copy(src.at[wid], src_v)
        pltpu.sync_copy(dst.at[wid], dst_v)
        if with_counts:
            @pl.loop(0, N_PAD // 16)
            def _(i):
                cnt_v[pl.ds(i * 16, 16)] = jnp.zeros((16,), jnp.float32)

        ones16 = jnp.ones((16,), jnp.float32)

        def count(j):
            for g in range(lane // 16):
                d16 = dst_v[j, pl.ds(g * 16, 16)]
                plsc.addupdate_scatter(cnt_v, [d16], ones16)

        for h, t_hbm in enumerate(tables):
            do_cnt = with_counts and h == 0
            pltpu.sync_copy(zeros, acc.at[pl.ds(row0, ROWS_PER_TILE)])
            if stage_table:
                pltpu.sync_copy(t_hbm.at[pl.ds(row0, ROWS_PER_TILE)],
                                table_sh.at[pl.ds(row0, ROWS_PER_TILE)])
                table = table_sh
            else:
                table = t_hbm
            plsc.subcore_barrier()

            def wait_gather(c, k):
                pltpu.make_async_copy(table.at[src_v.at[c]],
                                      bufs[k], gsem[k]).wait()

            def wait_scatter(c, k):
                pltpu.make_async_copy(bufs[k], acc.at[dst_v.at[c]],
                                      ssem[k]).wait()

            if async_scatter:
                pltpu.async_copy(table.at[src_v.at[0]], bufs[0], gsem[0])
                pltpu.async_copy(table.at[src_v.at[1]], bufs[1], gsem[1])
                main = ((nch - 2) // 3) * 3

                @pl.loop(0, main, step=3)
                def _(jj):
                    for b in range(3):
                        c = jj + b
                        k2 = (b + 2) % 3
                        wait_gather(c, b)

                        @pl.when(c >= 1)
                        def _():
                            wait_scatter(c - 1, (b + 2) % 3)

                        pltpu.async_copy(bufs[b], acc.at[dst_v.at[c]],
                                         ssem[b], add=True)
                        pltpu.async_copy(table.at[src_v.at[c + 2]],
                                         bufs[k2], gsem[k2])
                        if do_cnt:
                            count(c)
                for c in range(main, nch):
                    k = c % 3
                    wait_gather(c, k)
                    if c >= 1:
                        wait_scatter(c - 1, (c - 1) % 3)
                    pltpu.async_copy(bufs[k], acc.at[dst_v.at[c]],
                                     ssem[k], add=True)
                    if c + 2 < nch:
                        k2 = (c + 2) % 3
                        pltpu.async_copy(table.at[src_v.at[c + 2]],
                                         bufs[k2], gsem[k2])
                    if do_cnt:
                        count(c)
                wait_scatter(nch - 1, (nch - 1) % 3)
            else:
                pltpu.async_copy(table.at[src_v.at[0]], bufs[0], gsem[0])

                @pl.loop(0, nch, step=NBUF)
                def _(jj):
                    for b in range(NBUF):
                        nxt = (b + 1) % NBUF

                        @pl.when(jj + b + 1 < nch)
                        def _():
                            pltpu.async_copy(table.at[src_v.at[jj + b + 1]],
                                             bufs[nxt], gsem[nxt])

                        wait_gather(jj + b, b)
                        pltpu.sync_copy(bufs[b], acc.at[dst_v.at[jj + b]],
                                        add=True)
                        if do_cnt:
                            count(jj + b)

            plsc.subcore_barrier()
            pltpu.sync_copy(acc.at[pl.ds(row0, ROWS_PER_TILE)],
                            out.at[c, h, pl.ds(row0, ROWS_PER_TILE)])
        if with_counts:
            pltpu.sync_copy(cnt_v, cnt_out.at[wid])

    return agg


_sc_agg_l1 = _make_sc_agg(True)
_sc_agg_l2 = _make_sc_agg(False, lane=160, nch=64)

_BLK = 128
_GRID = N_PAD // _BLK


def _tc1_body(x_ref, s1a_ref, s1b_ref, cnt_ref,
              wl1_ref, wr1_ref, b1_ref, wl2_ref, h_ref, g2_ref, inv_ref):
    cnt = jnp.sum(cnt_ref[...], axis=0)
    inv = 1.0 / jnp.maximum(cnt, 1.0)
    inv_ref[...] = inv
    m = (s1a_ref[...] + s1b_ref[...]) * inv
    dn = (((1,), (1,)), ((), ()))
    pre = (lax.dot_general(m, wl1_ref[...], dn, preferred_element_type=jnp.float32)
           + lax.dot_general(x_ref[...], wr1_ref[...], dn,
                             preferred_element_type=jnp.float32)
           + b1_ref[...])
    h = jnp.maximum(pre, 0.0)
    h_ref[...] = h
    g2_ref[...] = lax.dot_general(h, wl2_ref[...], dn,
                                  preferred_element_type=jnp.float32)


def _tc2_body(h_ref, s2a_ref, s2b_ref, inv_ref, wr2_ref, b2_ref, o_ref):
    dn = (((1,), (1,)), ((), ()))
    o_ref[...] = ((s2a_ref[...] + s2b_ref[...]) * inv_ref[...]
                  + lax.dot_general(h_ref[...], wr2_ref[...], dn,
                                    preferred_element_type=jnp.float32)
                  + b2_ref[...])


def _row_spec(d):
    return pl.BlockSpec((_BLK, d), lambda i: (i, 0))


def _full_spec(r, c):
    return pl.BlockSpec((r, c), lambda i: (0, 0))


_tc1 = pl.pallas_call(
    _tc1_body,
    grid=(_GRID,),
    in_specs=[
        _row_spec(D_IN), _row_spec(D_IN), _row_spec(D_IN),
        pl.BlockSpec((NW, _BLK, 1), lambda i: (0, i, 0)),
        _full_spec(D_HID, D_IN), _full_spec(D_HID, D_IN), _full_spec(1, D_HID),
        _full_spec(D_OUT, D_HID),
    ],
    out_specs=[_row_spec(D_HID), _row_spec(D_OUT), _row_spec(1)],
    out_shape=[
        jax.ShapeDtypeStruct((N_PAD, D_HID), jnp.float32),
        jax.ShapeDtypeStruct((N_PAD, D_OUT), jnp.float32),
        jax.ShapeDtypeStruct((N_PAD, 1), jnp.float32),
    ],
)

_tc2 = pl.pallas_call(
    _tc2_body,
    grid=(_GRID,),
    in_specs=[
        _row_spec(D_HID), _row_spec(D_OUT), _row_spec(D_OUT), _row_spec(1),
        _full_spec(D_OUT, D_HID), _full_spec(1, D_OUT),
    ],
    out_specs=_row_spec(D_OUT),
    out_shape=jax.ShapeDtypeStruct((N_PAD, D_OUT), jnp.float32),
)


def _split(a):
    return tuple(a[:, i * HW:(i + 1) * HW] for i in range(D_IN // HW))


def _cat(p):
    planes = [p[i] for i in range(p.shape[0])]
    return planes[0] if len(planes) == 1 else jnp.concatenate(planes, axis=1)


def kernel(x, edge_index, Wl1, Wr1, b1, Wl2, Wr2, b2):
    src = edge_index[0]
    dst = edge_index[1]
    pad = E_PAD - E
    srcf = jnp.concatenate([src, jnp.zeros((pad,), jnp.int32)])
    dummy = N + jnp.arange(pad, dtype=jnp.int32) % (N_PAD - N)
    dstf = jnp.concatenate([dst, dummy])
    srcp = srcf.reshape(NW, NCH, LANE)
    dstp = dstf.reshape(NW, NCH, LANE)
    srcp2 = srcf.reshape(NW, 64, 160)
    dstp2 = dstf.reshape(NW, 64, 160)

    x_pad = jnp.zeros((N_PAD, D_IN), jnp.float32).at[:N].set(x)
    zeros = jnp.zeros((ROWS_PER_TILE, HW), jnp.float32)

    p1, cnt = _sc_agg_l1(*_split(x_pad), srcp, dstp, zeros)
    cnt3 = cnt.reshape(NW, N_PAD, 1)

    h, g2, inv = _tc1(x_pad, _cat(p1[0]), _cat(p1[1]), cnt3, Wl1, Wr1,
                      b1.reshape(1, D_HID), Wl2)

    p2 = _sc_agg_l2(*_split(g2), srcp2, dstp2, zeros)
    out = _tc2(h, _cat(p2[0]), _cat(p2[1]), inv, Wr2, b2.reshape(1, D_OUT))
    return out[:N]


# speedup vs baseline: 1.4835x; 1.4835x over previous
import functools

import jax
import jax.numpy as jnp
from jax import lax
from jax.experimental import pallas as pl
from jax.experimental.pallas import tpu as pltpu
from jax.experimental.pallas import tpu_sc as plsc

N = 10000
E = 320000
D_IN = 128
D_HID = 256
D_OUT = 128

NC = 2
NS = 16
NW = NC * NS
LANE = 128
HW = 64
N_PAD = 10240
ROWS_PER_TILE = N_PAD // NS
NCH = 80
E_PAD = NW * NCH * LANE
NBUF = 2


def _make_sc_agg(with_counts, W=HW, lane=LANE, nch=NCH, stage_table=True,
                 async_scatter=False):
    npass = D_IN // W
    mesh = plsc.VectorSubcoreMesh(core_axis_name="c", subcore_axis_name="s")
    out_type = [jax.ShapeDtypeStruct((N_PAD, W), jnp.float32)
                for _ in range(NC * npass)]
    if with_counts:
        out_type.append(jax.ShapeDtypeStruct((NW, N_PAD), jnp.float32))
    nbuf = 3 if async_scatter else NBUF
    scratch = [
        pltpu.VMEM((nch, lane), jnp.int32),
        pltpu.VMEM((nch, lane), jnp.int32),
        [pltpu.VMEM((lane, W), jnp.float32) for _ in range(nbuf)],
        pltpu.VMEM_SHARED((N_PAD, W), jnp.float32),
        [pltpu.SemaphoreType.DMA for _ in range(nbuf)],
        [pltpu.SemaphoreType.DMA for _ in range(nbuf)],
    ]
    if stage_table:
        scratch.append(pltpu.VMEM_SHARED((N_PAD, W), jnp.float32))
    if with_counts:
        scratch.append(pltpu.VMEM((N_PAD,), jnp.float32))

    @functools.partial(
        pl.kernel, out_type=out_type, mesh=mesh, scratch_types=scratch,
        compiler_params=pltpu.CompilerParams(needs_layout_passes=False,
                                             use_tc_tiling_on_sc=False))
    def agg(*args):
        tables = args[:npass]
        src, dst, zeros = args[npass:npass + 3]
        rest = args[npass + 3:]
        outs = rest[:NC * npass]
        rest = rest[NC * npass:]
        table_sh = None
        if with_counts and stage_table:
            cnt_out, src_v, dst_v, bufs, acc, gsem, ssem, table_sh, cnt_v = rest
        elif with_counts:
            cnt_out, src_v, dst_v, bufs, acc, gsem, ssem, cnt_v = rest
        elif stage_table:
            src_v, dst_v, bufs, acc, gsem, ssem, table_sh = rest
            cnt_out = cnt_v = None
        else:
            src_v, dst_v, bufs, acc, gsem, ssem = rest
            cnt_out = cnt_v = None
        c = lax.axis_index("c")
        s = lax.axis_index("s")
        wid = s * NC + c
        row0 = s * ROWS_PER_TILE
        pltpu.sync_copy(src.at[wid], src_v)
        pltpu.sync_copy(dst.at[wid], dst_v)
        if with_counts:
            @pl.loop(0, N_PAD // 16)
            def _(i):
                cnt_v[pl.ds(i * 16, 16)] = jnp.zeros((16,), jnp.float32)

        ones16 = jnp.ones((16,), jnp.float32)

        def count(j):
            for g in range(lane // 16):
                d16 = dst_v[j, pl.ds(g * 16, 16)]
                plsc.addupdate_scatter(cnt_v, [d16], ones16)

        for h, t_hbm in enumerate(tables):
            do_cnt = with_counts and h == 0
            pltpu.sync_copy(zeros, acc.at[pl.ds(row0, ROWS_PER_TILE)])
            if stage_table:
                pltpu.sync_copy(t_hbm.at[pl.ds(row0, ROWS_PER_TILE)],
                                table_sh.at[pl.ds(row0, ROWS_PER_TILE)])
                table = table_sh
            else:
                table = t_hbm
            plsc.subcore_barrier()

            def wait_gather(c, k):
                pltpu.make_async_copy(table.at[src_v.at[c]],
                                      bufs[k], gsem[k]).wait()

            def wait_scatter(c, k):
                pltpu.make_async_copy(bufs[k], acc.at[dst_v.at[c]],
                                      ssem[k]).wait()

            if async_scatter:
                pltpu.async_copy(table.at[src_v.at[0]], bufs[0], gsem[0])
                pltpu.async_copy(table.at[src_v.at[1]], bufs[1], gsem[1])
                main = ((nch - 2) // 3) * 3

                @pl.loop(0, main, step=3)
                def _(jj):
                    for b in range(3):
                        c = jj + b
                        k2 = (b + 2) % 3
                        wait_gather(c, b)

                        @pl.when(c >= 1)
                        def _():
                            wait_scatter(c - 1, (b + 2) % 3)

                        pltpu.async_copy(bufs[b], acc.at[dst_v.at[c]],
                                         ssem[b], add=True)
                        pltpu.async_copy(table.at[src_v.at[c + 2]],
                                         bufs[k2], gsem[k2])
                        if do_cnt:
                            count(c)
                for c in range(main, nch):
                    k = c % 3
                    wait_gather(c, k)
                    if c >= 1:
                        wait_scatter(c - 1, (c - 1) % 3)
                    pltpu.async_copy(bufs[k], acc.at[dst_v.at[c]],
                                     ssem[k], add=True)
                    if c + 2 < nch:
                        k2 = (c + 2) % 3
                        pltpu.async_copy(table.at[src_v.at[c + 2]],
                                         bufs[k2], gsem[k2])
                    if do_cnt:
                        count(c)
                wait_scatter(nch - 1, (nch - 1) % 3)
            else:
                pltpu.async_copy(table.at[src_v.at[0]], bufs[0], gsem[0])

                @pl.loop(0, nch, step=NBUF)
                def _(jj):
                    for b in range(NBUF):
                        nxt = (b + 1) % NBUF

                        @pl.when(jj + b + 1 < nch)
                        def _():
                            pltpu.async_copy(table.at[src_v.at[jj + b + 1]],
                                             bufs[nxt], gsem[nxt])

                        wait_gather(jj + b, b)
                        pltpu.sync_copy(bufs[b], acc.at[dst_v.at[jj + b]],
                                        add=True)
                        if do_cnt:
                            count(jj + b)

            plsc.subcore_barrier()
            for cc in range(NC):
                @pl.when(c == cc)
                def _():
                    pltpu.sync_copy(
                        acc.at[pl.ds(row0, ROWS_PER_TILE)],
                        outs[cc * npass + h].at[pl.ds(row0, ROWS_PER_TILE)])
        if with_counts:
            pltpu.sync_copy(cnt_v, cnt_out.at[wid])

    return agg


_sc_agg_l1 = _make_sc_agg(True)
_sc_agg_l2 = _make_sc_agg(False)

_BLK = 128
_GRID = N_PAD // _BLK


def _scale_rows(inv_lane, s):
    eye = jnp.eye(_BLK, dtype=jnp.float32)
    dn = (((1,), (0,)), ((), ()))
    return lax.dot_general(eye * inv_lane, s, dn,
                           preferred_element_type=jnp.float32)


def _tc1_body(x_ref, p00_ref, p01_ref, p10_ref, p11_ref, cnt_ref,
              wl1_ref, wr1_ref, b1_ref, wl2_ref,
              h_ref, g2lo_ref, g2hi_ref, inv_ref):
    cnt = jnp.sum(cnt_ref[...], axis=0).reshape(1, _BLK)
    inv = 1.0 / jnp.maximum(cnt, 1.0)
    inv_ref[...] = inv.reshape(1, 1, _BLK)
    s1 = jnp.concatenate([p00_ref[...] + p10_ref[...],
                          p01_ref[...] + p11_ref[...]], axis=1)
    m = _scale_rows(inv, s1)
    dn = (((1,), (1,)), ((), ()))
    pre = (lax.dot_general(m, wl1_ref[...], dn, preferred_element_type=jnp.float32)
           + lax.dot_general(x_ref[...], wr1_ref[...], dn,
                             preferred_element_type=jnp.float32)
           + b1_ref[...])
    h = jnp.maximum(pre, 0.0)
    h_ref[...] = h
    g2 = lax.dot_general(h, wl2_ref[...], dn, preferred_element_type=jnp.float32)
    g2lo_ref[...] = g2[:, :HW]
    g2hi_ref[...] = g2[:, HW:]


def _tc2_body(h_ref, q00_ref, q01_ref, q10_ref, q11_ref, inv_ref,
              wr2_ref, b2_ref, o_ref):
    s2 = jnp.concatenate([q00_ref[...] + q10_ref[...],
                          q01_ref[...] + q11_ref[...]], axis=1)
    inv = inv_ref[...].reshape(1, _BLK)
    dn = (((1,), (1,)), ((), ()))
    o_ref[...] = (_scale_rows(inv, s2)
                  + lax.dot_general(h_ref[...], wr2_ref[...], dn,
                                    preferred_element_type=jnp.float32)
                  + b2_ref[...])


def _row_spec(d):
    return pl.BlockSpec((_BLK, d), lambda i: (i, 0))


def _full_spec(r, c):
    return pl.BlockSpec((r, c), lambda i: (0, 0))


_CNT_SPEC = pl.BlockSpec((NW, _BLK), lambda i: (0, i))
_INV_SPEC = pl.BlockSpec((1, 1, _BLK), lambda i: (i, 0, 0))

_tc1 = pl.pallas_call(
    _tc1_body,
    grid=(_GRID,),
    in_specs=[
        _row_spec(D_IN),
        _row_spec(HW), _row_spec(HW), _row_spec(HW), _row_spec(HW),
        _CNT_SPEC,
        _full_spec(D_HID, D_IN), _full_spec(D_HID, D_IN), _full_spec(1, D_HID),
        _full_spec(D_OUT, D_HID),
    ],
    out_specs=[_row_spec(D_HID), _row_spec(HW), _row_spec(HW), _INV_SPEC],
    out_shape=[
        jax.ShapeDtypeStruct((N_PAD, D_HID), jnp.float32),
        jax.ShapeDtypeStruct((N_PAD, HW), jnp.float32),
        jax.ShapeDtypeStruct((N_PAD, HW), jnp.float32),
        jax.ShapeDtypeStruct((_GRID, 1, _BLK), jnp.float32),
    ],
)

_tc2 = pl.pallas_call(
    _tc2_body,
    grid=(_GRID,),
    in_specs=[
        _row_spec(D_HID),
        _row_spec(HW), _row_spec(HW), _row_spec(HW), _row_spec(HW),
        _INV_SPEC,
        _full_spec(D_OUT, D_HID), _full_spec(1, D_OUT),
    ],
    out_specs=_row_spec(D_OUT),
    out_shape=jax.ShapeDtypeStruct((N_PAD, D_OUT), jnp.float32),
)


def _split(a):
    return tuple(a[:, i * HW:(i + 1) * HW] for i in range(D_IN // HW))


def kernel(x, edge_index, Wl1, Wr1, b1, Wl2, Wr2, b2):
    src = edge_index[0]
    dst = edge_index[1]
    pad = E_PAD - E
    srcf = jnp.concatenate([src, jnp.zeros((pad,), jnp.int32)])
    dummy = N + jnp.arange(pad, dtype=jnp.int32) % (N_PAD - N)
    dstf = jnp.concatenate([dst, dummy])
    srcp = srcf.reshape(NW, NCH, LANE)
    dstp = dstf.reshape(NW, NCH, LANE)

    x_pad = jnp.zeros((N_PAD, D_IN), jnp.float32).at[:N].set(x)
    zeros = jnp.zeros((ROWS_PER_TILE, HW), jnp.float32)

    p00, p01, p10, p11, cnt = _sc_agg_l1(*_split(x_pad), srcp, dstp, zeros)

    h, g2lo, g2hi, inv = _tc1(x_pad, p00, p01, p10, p11, cnt, Wl1, Wr1,
                              b1.reshape(1, D_HID), Wl2)

    q00, q01, q10, q11 = _sc_agg_l2(g2lo, g2hi, srcp, dstp, zeros)
    out = _tc2(h, q00, q01, q10, q11, inv, Wr2, b2.reshape(1, D_OUT))
    return out[:N]


# speedup vs baseline: 1.6289x; 1.0980x over previous
import functools

import jax
import jax.numpy as jnp
from jax import lax
from jax.experimental import pallas as pl
from jax.experimental.pallas import tpu as pltpu
from jax.experimental.pallas import tpu_sc as plsc

N = 10000
E = 320000
D_IN = 128
D_HID = 256
D_OUT = 128

NC = 2
NS = 16
NW = NC * NS
LANE = 128
HW = 64
N_PAD = 10240
ROWS_PER_TILE = N_PAD // NS
NCH = 80
E_PAD = NW * NCH * LANE
NBUF = 2


def _make_sc_agg(with_counts, W=HW, lane=LANE, nch=NCH, stage_table=True,
                 async_scatter=False):
    npass = D_IN // W
    mesh = plsc.VectorSubcoreMesh(core_axis_name="c", subcore_axis_name="s")
    out_type = [jax.ShapeDtypeStruct((N_PAD, W), jnp.float32)
                for _ in range(NC * npass)]
    if with_counts:
        out_type.append(jax.ShapeDtypeStruct((NW, N_PAD), jnp.float32))
    nbuf = 3 if async_scatter else NBUF
    scratch = [
        pltpu.VMEM((nch, lane), jnp.int32),
        pltpu.VMEM((nch, lane), jnp.int32),
        [pltpu.VMEM((lane, W), jnp.float32) for _ in range(nbuf)],
        pltpu.VMEM_SHARED((N_PAD, W), jnp.float32),
        [pltpu.SemaphoreType.DMA for _ in range(nbuf)],
        [pltpu.SemaphoreType.DMA for _ in range(nbuf)],
    ]
    if stage_table:
        scratch.append(pltpu.VMEM_SHARED((N_PAD, W), jnp.float32))
    if with_counts:
        scratch.append(pltpu.VMEM((N_PAD,), jnp.float32))

    @functools.partial(
        pl.kernel, out_type=out_type, mesh=mesh, scratch_types=scratch,
        compiler_params=pltpu.CompilerParams(needs_layout_passes=False,
                                             use_tc_tiling_on_sc=False))
    def agg(*args):
        tables = args[:npass]
        src, dst, zeros = args[npass:npass + 3]
        rest = args[npass + 3:]
        outs = rest[:NC * npass]
        rest = rest[NC * npass:]
        table_sh = None
        if with_counts and stage_table:
            cnt_out, src_v, dst_v, bufs, acc, gsem, ssem, table_sh, cnt_v = rest
        elif with_counts:
            cnt_out, src_v, dst_v, bufs, acc, gsem, ssem, cnt_v = rest
        elif stage_table:
            src_v, dst_v, bufs, acc, gsem, ssem, table_sh = rest
            cnt_out = cnt_v = None
        else:
            src_v, dst_v, bufs, acc, gsem, ssem = rest
            cnt_out = cnt_v = None
        c = lax.axis_index("c")
        s = lax.axis_index("s")
        wid = s * NC + c
        row0 = s * ROWS_PER_TILE
        pltpu.sync_copy(src.at[wid], src_v)
        pltpu.sync_copy(dst.at[wid], dst_v)
        if with_counts:
            @pl.loop(0, N_PAD // 16)
            def _(i):
                cnt_v[pl.ds(i * 16, 16)] = jnp.zeros((16,), jnp.float32)

        ones16 = jnp.ones((16,), jnp.float32)

        def count(j):
            for g in range(lane // 16):
                d16 = dst_v[j, pl.ds(g * 16, 16)]
                plsc.addupdate_scatter(cnt_v, [d16], ones16)

        for h, t_hbm in enumerate(tables):
            do_cnt = with_counts and h == 0
            pltpu.sync_copy(zeros, acc.at[pl.ds(row0, ROWS_PER_TILE)])
            if stage_table:
                pltpu.sync_copy(t_hbm.at[pl.ds(row0, ROWS_PER_TILE)],
                                table_sh.at[pl.ds(row0, ROWS_PER_TILE)])
                table = table_sh
            else:
                table = t_hbm
            plsc.subcore_barrier()

            def wait_gather(c, k):
                pltpu.make_async_copy(table.at[src_v.at[c]],
                                      bufs[k], gsem[k]).wait()

            def wait_scatter(c, k):
                pltpu.make_async_copy(bufs[k], acc.at[dst_v.at[c]],
                                      ssem[k]).wait()

            if async_scatter:
                pltpu.async_copy(table.at[src_v.at[0]], bufs[0], gsem[0])
                pltpu.async_copy(table.at[src_v.at[1]], bufs[1], gsem[1])
                main = ((nch - 2) // 3) * 3

                @pl.loop(0, main, step=3)
                def _(jj):
                    for b in range(3):
                        c = jj + b
                        k2 = (b + 2) % 3
                        wait_gather(c, b)

                        @pl.when(c >= 1)
                        def _():
                            wait_scatter(c - 1, (b + 2) % 3)

                        pltpu.async_copy(bufs[b], acc.at[dst_v.at[c]],
                                         ssem[b], add=True)
                        pltpu.async_copy(table.at[src_v.at[c + 2]],
                                         bufs[k2], gsem[k2])
                        if do_cnt:
                            count(c)
                for c in range(main, nch):
                    k = c % 3
                    wait_gather(c, k)
                    if c >= 1:
                        wait_scatter(c - 1, (c - 1) % 3)
                    pltpu.async_copy(bufs[k], acc.at[dst_v.at[c]],
                                     ssem[k], add=True)
                    if c + 2 < nch:
                        k2 = (c + 2) % 3
                        pltpu.async_copy(table.at[src_v.at[c + 2]],
                                         bufs[k2], gsem[k2])
                    if do_cnt:
                        count(c)
                wait_scatter(nch - 1, (nch - 1) % 3)
            else:
                pltpu.async_copy(table.at[src_v.at[0]], bufs[0], gsem[0])

                @pl.loop(0, nch, step=NBUF)
                def _(jj):
                    for b in range(NBUF):
                        nxt = (b + 1) % NBUF

                        @pl.when(jj + b + 1 < nch)
                        def _():
                            pltpu.async_copy(table.at[src_v.at[jj + b + 1]],
                                             bufs[nxt], gsem[nxt])

                        wait_gather(jj + b, b)
                        pltpu.sync_copy(bufs[b], acc.at[dst_v.at[jj + b]],
                                        add=True)
                        if do_cnt:
                            count(jj + b)

            plsc.subcore_barrier()
            for cc in range(NC):
                @pl.when(c == cc)
                def _():
                    pltpu.sync_copy(
                        acc.at[pl.ds(row0, ROWS_PER_TILE)],
                        outs[cc * npass + h].at[pl.ds(row0, ROWS_PER_TILE)])
        if with_counts:
            pltpu.sync_copy(cnt_v, cnt_out.at[wid])

    return agg


_sc_agg_l1 = _make_sc_agg(True)
_sc_agg_l2 = _make_sc_agg(False)

_BLK = 256
_GRID = N_PAD // _BLK


def _scale_rows(inv_lane, s):
    eye = jnp.eye(_BLK, dtype=jnp.float32)
    dn = (((1,), (0,)), ((), ()))
    return lax.dot_general(eye * inv_lane, s, dn,
                           preferred_element_type=jnp.float32)


def _tc_pre1_body(x_ref, wr1_ref, b1_ref, pre_ref):
    dn = (((1,), (1,)), ((), ()))
    pre_ref[...] = lax.dot_general(x_ref[...], wr1_ref[...], dn,
                                   preferred_element_type=jnp.float32) + b1_ref[...]


def _tc_pre2_body(h_ref, wr2_ref, b2_ref, o2_ref):
    dn = (((1,), (1,)), ((), ()))
    o2_ref[...] = lax.dot_general(h_ref[...], wr2_ref[...], dn,
                                  preferred_element_type=jnp.float32) + b2_ref[...]


def _tc1_body(pre_ref, p00_ref, p01_ref, p10_ref, p11_ref, cnt_ref,
              wl1_ref, wl2_ref,
              h_ref, g2lo_ref, g2hi_ref, inv_ref):
    cnt = jnp.sum(cnt_ref[...], axis=0).reshape(1, _BLK)
    inv = 1.0 / jnp.maximum(cnt, 1.0)
    inv_ref[...] = inv.reshape(1, 1, _BLK)
    s1 = jnp.concatenate([p00_ref[...] + p10_ref[...],
                          p01_ref[...] + p11_ref[...]], axis=1)
    m = _scale_rows(inv, s1)
    dn = (((1,), (1,)), ((), ()))
    pre = (lax.dot_general(m, wl1_ref[...], dn, preferred_element_type=jnp.float32)
           + pre_ref[...])
    h = jnp.maximum(pre, 0.0)
    h_ref[...] = h
    g2 = lax.dot_general(h, wl2_ref[...], dn, preferred_element_type=jnp.float32)
    g2lo_ref[...] = g2[:, :HW]
    g2hi_ref[...] = g2[:, HW:]


def _tc2_body(q00_ref, q01_ref, q10_ref, q11_ref, inv_ref, o2_ref, o_ref):
    s2 = jnp.concatenate([q00_ref[...] + q10_ref[...],
                          q01_ref[...] + q11_ref[...]], axis=1)
    inv = inv_ref[...].reshape(1, _BLK)
    o_ref[...] = _scale_rows(inv, s2) + o2_ref[...]


def _row_spec(d):
    return pl.BlockSpec((_BLK, d), lambda i: (i, 0))


def _full_spec(r, c):
    return pl.BlockSpec((r, c), lambda i: (0, 0))


_CNT_SPEC = pl.BlockSpec((NW, _BLK), lambda i: (0, i))
_INV_SPEC = pl.BlockSpec((1, 1, _BLK), lambda i: (i, 0, 0))

_tc_pre1 = pl.pallas_call(
    _tc_pre1_body,
    grid=(_GRID,),
    in_specs=[_row_spec(D_IN), _full_spec(D_HID, D_IN), _full_spec(1, D_HID)],
    out_specs=_row_spec(D_HID),
    out_shape=jax.ShapeDtypeStruct((N_PAD, D_HID), jnp.float32),
)

_tc_pre2 = pl.pallas_call(
    _tc_pre2_body,
    grid=(_GRID,),
    in_specs=[_row_spec(D_HID), _full_spec(D_OUT, D_HID), _full_spec(1, D_OUT)],
    out_specs=_row_spec(D_OUT),
    out_shape=jax.ShapeDtypeStruct((N_PAD, D_OUT), jnp.float32),
)

_tc1 = pl.pallas_call(
    _tc1_body,
    grid=(_GRID,),
    in_specs=[
        _row_spec(D_HID),
        _row_spec(HW), _row_spec(HW), _row_spec(HW), _row_spec(HW),
        _CNT_SPEC,
        _full_spec(D_HID, D_IN),
        _full_spec(D_OUT, D_HID),
    ],
    out_specs=[_row_spec(D_HID), _row_spec(HW), _row_spec(HW), _INV_SPEC],
    out_shape=[
        jax.ShapeDtypeStruct((N_PAD, D_HID), jnp.float32),
        jax.ShapeDtypeStruct((N_PAD, HW), jnp.float32),
        jax.ShapeDtypeStruct((N_PAD, HW), jnp.float32),
        jax.ShapeDtypeStruct((_GRID, 1, _BLK), jnp.float32),
    ],
)

_tc2 = pl.pallas_call(
    _tc2_body,
    grid=(_GRID,),
    in_specs=[
        _row_spec(HW), _row_spec(HW), _row_spec(HW), _row_spec(HW),
        _INV_SPEC,
        _row_spec(D_OUT),
    ],
    out_specs=_row_spec(D_OUT),
    out_shape=jax.ShapeDtypeStruct((N_PAD, D_OUT), jnp.float32),
)


def _split(a):
    return tuple(a[:, i * HW:(i + 1) * HW] for i in range(D_IN // HW))


def kernel(x, edge_index, Wl1, Wr1, b1, Wl2, Wr2, b2):
    src = edge_index[0]
    dst = edge_index[1]
    pad = E_PAD - E
    srcf = jnp.concatenate([src, jnp.zeros((pad,), jnp.int32)])
    dummy = N + jnp.arange(pad, dtype=jnp.int32) % (N_PAD - N)
    dstf = jnp.concatenate([dst, dummy])
    srcp = srcf.reshape(NW, NCH, LANE)
    dstp = dstf.reshape(NW, NCH, LANE)

    x_pad = jnp.zeros((N_PAD, D_IN), jnp.float32).at[:N].set(x)
    zeros = jnp.zeros((ROWS_PER_TILE, HW), jnp.float32)

    pre1 = _tc_pre1(x_pad, Wr1, b1.reshape(1, D_HID))
    p00, p01, p10, p11, cnt = _sc_agg_l1(*_split(x_pad), srcp, dstp, zeros)

    h, g2lo, g2hi, inv = _tc1(pre1, p00, p01, p10, p11, cnt, Wl1, Wl2)

    o2 = _tc_pre2(h, Wr2, b2.reshape(1, D_OUT))
    q00, q01, q10, q11 = _sc_agg_l2(g2lo, g2hi, srcp, dstp, zeros)
    out = _tc2(q00, q01, q10, q11, inv, o2)
    return out[:N]


# speedup vs baseline: 1.8454x; 1.1329x over previous
import functools

import jax
import jax.numpy as jnp
from jax import lax
from jax.experimental import pallas as pl
from jax.experimental.pallas import tpu as pltpu
from jax.experimental.pallas import tpu_sc as plsc

N = 10000
E = 320000
D_IN = 128
D_HID = 256
D_OUT = 128

NC = 2
NS = 16
NW = NC * NS
LANE = 128
HW = 64
N_PAD = 10240
ROWS_PER_TILE = N_PAD // NS
NCH = 80
E_PAD = NW * NCH * LANE
NBUF = 2


def _make_sc_agg(with_counts, W=HW, lane=LANE, nch=NCH):
    npass = D_IN // W
    mesh = plsc.VectorSubcoreMesh(core_axis_name="c", subcore_axis_name="s")
    out_type = [jax.ShapeDtypeStruct((N_PAD, D_IN), jnp.float32)
                for _ in range(NC)]
    if with_counts:
        out_type.append(jax.ShapeDtypeStruct((NW, N_PAD), jnp.float32))
    scratch = [
        pltpu.VMEM((nch, lane), jnp.int32),
        pltpu.VMEM((nch, lane), jnp.int32),
        [pltpu.VMEM((lane, W), jnp.float32) for _ in range(NBUF)],
        pltpu.VMEM_SHARED((N_PAD, W), jnp.float32),
        pltpu.VMEM_SHARED((N_PAD, W), jnp.float32),
        [pltpu.SemaphoreType.DMA for _ in range(NBUF)],
    ]
    if with_counts:
        scratch.append(pltpu.VMEM((N_PAD,), jnp.float32))

    @functools.partial(
        pl.kernel, out_type=out_type, mesh=mesh, scratch_types=scratch,
        compiler_params=pltpu.CompilerParams(needs_layout_passes=False,
                                             use_tc_tiling_on_sc=False))
    def agg(*args):
        table_hbm, src, dst, zeros = args[:4]
        rest = args[4:]
        outs = rest[:NC]
        rest = rest[NC:]
        if with_counts:
            cnt_out, src_v, dst_v, bufs, acc, table_sh, gsem, cnt_v = rest
        else:
            src_v, dst_v, bufs, acc, table_sh, gsem = rest
            cnt_out = cnt_v = None
        c = lax.axis_index("c")
        s = lax.axis_index("s")
        wid = s * NC + c
        row0 = s * ROWS_PER_TILE
        rows = pl.ds(row0, ROWS_PER_TILE)
        pltpu.sync_copy(src.at[wid], src_v)
        pltpu.sync_copy(dst.at[wid], dst_v)
        if with_counts:
            @pl.loop(0, N_PAD // 16)
            def _(i):
                cnt_v[pl.ds(i * 16, 16)] = jnp.zeros((16,), jnp.float32)

        ones16 = jnp.ones((16,), jnp.float32)

        def count(j):
            for g in range(lane // 16):
                d16 = dst_v[j, pl.ds(g * 16, 16)]
                plsc.addupdate_scatter(cnt_v, [d16], ones16)

        for h in range(npass):
            cols = pl.ds(h * W, W)
            do_cnt = with_counts and h == 0
            pltpu.sync_copy(zeros, acc.at[rows])
            pltpu.sync_copy(table_hbm.at[rows, cols], table_sh.at[rows])
            plsc.subcore_barrier()

            pltpu.async_copy(table_sh.at[src_v.at[0]], bufs[0], gsem[0])

            @pl.loop(0, nch, step=NBUF)
            def _(jj):
                for b in range(NBUF):
                    nxt = (b + 1) % NBUF

                    @pl.when(jj + b + 1 < nch)
                    def _():
                        pltpu.async_copy(table_sh.at[src_v.at[jj + b + 1]],
                                         bufs[nxt], gsem[nxt])

                    pltpu.make_async_copy(table_sh.at[src_v.at[jj + b]],
                                          bufs[b], gsem[b]).wait()
                    pltpu.sync_copy(bufs[b], acc.at[dst_v.at[jj + b]],
                                    add=True)
                    if do_cnt:
                        count(jj + b)

            plsc.subcore_barrier()
            for cc in range(NC):
                @pl.when(c == cc)
                def _():
                    pltpu.sync_copy(acc.at[rows], outs[cc].at[rows, cols])
        if with_counts:
            pltpu.sync_copy(cnt_v, cnt_out.at[wid])

    return agg


_sc_agg_l1 = _make_sc_agg(True)
_sc_agg_l2 = _make_sc_agg(False)

_BLK = 256
_GRID = N_PAD // _BLK


def _scale_rows(inv_lane, s):
    eye = jnp.eye(_BLK, dtype=jnp.float32)
    dn = (((1,), (0,)), ((), ()))
    return lax.dot_general(eye * inv_lane, s, dn,
                           preferred_element_type=jnp.float32)


def _tc_pre1_body(x_ref, wr1_ref, b1_ref, pre_ref):
    dn = (((1,), (1,)), ((), ()))
    pre_ref[...] = lax.dot_general(x_ref[...], wr1_ref[...], dn,
                                   preferred_element_type=jnp.float32) + b1_ref[...]


def _tc_pre2_body(h_ref, wr2_ref, b2_ref, o2_ref):
    dn = (((1,), (1,)), ((), ()))
    o2_ref[...] = lax.dot_general(h_ref[...], wr2_ref[...], dn,
                                  preferred_element_type=jnp.float32) + b2_ref[...]


def _tc1_body(pre_ref, pa_ref, pb_ref, cnt_ref, wl1_ref, wl2_ref,
              h_ref, g2_ref, inv_ref):
    cnt = jnp.sum(cnt_ref[...], axis=0).reshape(1, _BLK)
    inv = 1.0 / jnp.maximum(cnt, 1.0)
    inv_ref[...] = inv.reshape(1, 1, _BLK)
    m = _scale_rows(inv, pa_ref[...] + pb_ref[...])
    dn = (((1,), (1,)), ((), ()))
    pre = (lax.dot_general(m, wl1_ref[...], dn, preferred_element_type=jnp.float32)
           + pre_ref[...])
    h = jnp.maximum(pre, 0.0)
    h_ref[...] = h
    g2_ref[...] = lax.dot_general(h, wl2_ref[...], dn,
                                  preferred_element_type=jnp.float32)


def _tc2_body(qa_ref, qb_ref, inv_ref, o2_ref, o_ref):
    inv = inv_ref[...].reshape(1, _BLK)
    o_ref[...] = _scale_rows(inv, qa_ref[...] + qb_ref[...]) + o2_ref[...]


def _row_spec(d):
    return pl.BlockSpec((_BLK, d), lambda i: (i, 0))


def _full_spec(r, c):
    return pl.BlockSpec((r, c), lambda i: (0, 0))


_CNT_SPEC = pl.BlockSpec((NW, _BLK), lambda i: (0, i))
_INV_SPEC = pl.BlockSpec((1, 1, _BLK), lambda i: (i, 0, 0))

_tc_pre1 = pl.pallas_call(
    _tc_pre1_body,
    grid=(_GRID,),
    in_specs=[_row_spec(D_IN), _full_spec(D_HID, D_IN), _full_spec(1, D_HID)],
    out_specs=_row_spec(D_HID),
    out_shape=jax.ShapeDtypeStruct((N_PAD, D_HID), jnp.float32),
)

_tc_pre2 = pl.pallas_call(
    _tc_pre2_body,
    grid=(_GRID,),
    in_specs=[_row_spec(D_HID), _full_spec(D_OUT, D_HID), _full_spec(1, D_OUT)],
    out_specs=_row_spec(D_OUT),
    out_shape=jax.ShapeDtypeStruct((N_PAD, D_OUT), jnp.float32),
)

_tc1 = pl.pallas_call(
    _tc1_body,
    grid=(_GRID,),
    in_specs=[
        _row_spec(D_HID), _row_spec(D_IN), _row_spec(D_IN),
        _CNT_SPEC,
        _full_spec(D_HID, D_IN),
        _full_spec(D_OUT, D_HID),
    ],
    out_specs=[_row_spec(D_HID), _row_spec(D_OUT), _INV_SPEC],
    out_shape=[
        jax.ShapeDtypeStruct((N_PAD, D_HID), jnp.float32),
        jax.ShapeDtypeStruct((N_PAD, D_OUT), jnp.float32),
        jax.ShapeDtypeStruct((_GRID, 1, _BLK), jnp.float32),
    ],
)

_tc2 = pl.pallas_call(
    _tc2_body,
    grid=(_GRID,),
    in_specs=[
        _row_spec(D_OUT), _row_spec(D_OUT),
        _INV_SPEC,
        _row_spec(D_OUT),
    ],
    out_specs=_row_spec(D_OUT),
    out_shape=jax.ShapeDtypeStruct((N_PAD, D_OUT), jnp.float32),
)


def kernel(x, edge_index, Wl1, Wr1, b1, Wl2, Wr2, b2):
    src = edge_index[0]
    dst = edge_index[1]
    pad = E_PAD - E
    srcf = jnp.concatenate([src, jnp.zeros((pad,), jnp.int32)])
    dummy = N + jnp.arange(pad, dtype=jnp.int32) % (N_PAD - N)
    dstf = jnp.concatenate([dst, dummy])
    srcp = srcf.reshape(NW, NCH, LANE)
    dstp = dstf.reshape(NW, NCH, LANE)

    x_pad = jnp.zeros((N_PAD, D_IN), jnp.float32).at[:N].set(x)
    zeros = jnp.zeros((ROWS_PER_TILE, HW), jnp.float32)

    pre1 = _tc_pre1(x_pad, Wr1, b1.reshape(1, D_HID))
    pa, pb, cnt = _sc_agg_l1(x_pad, srcp, dstp, zeros)

    h, g2, inv = _tc1(pre1, pa, pb, cnt, Wl1, Wl2)

    o2 = _tc_pre2(h, Wr2, b2.reshape(1, D_OUT))
    qa, qb = _sc_agg_l2(g2, srcp, dstp, zeros)
    out = _tc2(qa, qb, inv, o2)
    return out[:N]


# speedup vs baseline: 1.9263x; 1.0438x over previous
import functools

import jax
import jax.numpy as jnp
from jax import lax
from jax.experimental import pallas as pl
from jax.experimental.pallas import tpu as pltpu
from jax.experimental.pallas import tpu_sc as plsc

N = 10000
E = 320000
D_IN = 128
D_HID = 256
D_OUT = 128

NC = 2
NS = 16
NW = NC * NS
LANE = 128
HW = 64
N_PAD = 10240
ROWS_PER_TILE = N_PAD // NS
NCH = 80
E_PAD = NW * NCH * LANE
EPT = E_PAD // NW
REAL_LAST = E - (NW - 1) * EPT
NBUF = 2


def _make_sc_agg(with_counts, W=HW, lane=LANE, nch=NCH):
    npass = D_IN // W
    mesh = plsc.VectorSubcoreMesh(core_axis_name="c", subcore_axis_name="s")
    out_type = [jax.ShapeDtypeStruct((N_PAD, D_IN), jnp.float32)
                for _ in range(NC)]
    if with_counts:
        out_type.append(jax.ShapeDtypeStruct((NW, N_PAD), jnp.float32))
    scratch = [
        pltpu.VMEM((EPT,), jnp.int32),
        pltpu.VMEM((EPT,), jnp.int32),
        [pltpu.VMEM((lane, W), jnp.float32) for _ in range(NBUF)],
        pltpu.VMEM_SHARED((N_PAD, W), jnp.float32),
        pltpu.VMEM_SHARED((N_PAD, W), jnp.float32),
        [pltpu.SemaphoreType.DMA for _ in range(NBUF)],
    ]
    if with_counts:
        scratch.append(pltpu.VMEM((N_PAD,), jnp.float32))

    @functools.partial(
        pl.kernel, out_type=out_type, mesh=mesh, scratch_types=scratch,
        compiler_params=pltpu.CompilerParams(needs_layout_passes=False,
                                             use_tc_tiling_on_sc=False))
    def agg(*args):
        table_hbm, edge, zeros = args[:3]
        rest = args[3:]
        outs = rest[:NC]
        rest = rest[NC:]
        if with_counts:
            cnt_out, src_v, dst_v, bufs, acc, table_sh, gsem, cnt_v = rest
        else:
            src_v, dst_v, bufs, acc, table_sh, gsem = rest
            cnt_out = cnt_v = None
        c = lax.axis_index("c")
        s = lax.axis_index("s")
        wid = s * NC + c
        row0 = s * ROWS_PER_TILE
        rows = pl.ds(row0, ROWS_PER_TILE)
        base = wid * EPT

        @pl.when(wid < NW - 1)
        def _():
            pltpu.sync_copy(edge.at[0, pl.ds(base, EPT)], src_v)
            pltpu.sync_copy(edge.at[1, pl.ds(base, EPT)], dst_v)

        @pl.when(wid == NW - 1)
        def _():
            pltpu.sync_copy(edge.at[0, pl.ds(E - REAL_LAST, REAL_LAST)],
                            src_v.at[pl.ds(0, REAL_LAST)])
            pltpu.sync_copy(edge.at[1, pl.ds(E - REAL_LAST, REAL_LAST)],
                            dst_v.at[pl.ds(0, REAL_LAST)])
            lanes = lax.iota(jnp.int32, 16)

            @pl.loop(0, (EPT - REAL_LAST) // 16)
            def _(i):
                off = REAL_LAST + i * 16
                src_v[pl.ds(off, 16)] = jnp.zeros((16,), jnp.int32)
                dst_v[pl.ds(off, 16)] = N + lax.bitwise_and(off + lanes, 127)

        if with_counts:
            @pl.loop(0, N_PAD // 16)
            def _(i):
                cnt_v[pl.ds(i * 16, 16)] = jnp.zeros((16,), jnp.float32)

        ones16 = jnp.ones((16,), jnp.float32)

        def count(j):
            for g in range(lane // 16):
                d16 = dst_v[pl.ds(j * lane + g * 16, 16)]
                plsc.addupdate_scatter(cnt_v, [d16], ones16)

        for h in range(npass):
            cols = pl.ds(h * W, W)
            do_cnt = with_counts and h == 0
            pltpu.sync_copy(zeros, acc.at[rows])
            pltpu.sync_copy(table_hbm.at[rows, cols], table_sh.at[rows])
            plsc.subcore_barrier()

            def sidx(j):
                return src_v.at[pl.ds(j * lane, lane)]

            def didx(j):
                return dst_v.at[pl.ds(j * lane, lane)]

            pltpu.async_copy(table_sh.at[sidx(0)], bufs[0], gsem[0])

            @pl.loop(0, nch, step=NBUF)
            def _(jj):
                for b in range(NBUF):
                    nxt = (b + 1) % NBUF

                    @pl.when(jj + b + 1 < nch)
                    def _():
                        pltpu.async_copy(table_sh.at[sidx(jj + b + 1)],
                                         bufs[nxt], gsem[nxt])

                    pltpu.make_async_copy(table_sh.at[sidx(jj + b)],
                                          bufs[b], gsem[b]).wait()
                    pltpu.sync_copy(bufs[b], acc.at[didx(jj + b)], add=True)
                    if do_cnt:
                        count(jj + b)

            plsc.subcore_barrier()
            for cc in range(NC):
                @pl.when(c == cc)
                def _():
                    pltpu.sync_copy(acc.at[rows], outs[cc].at[rows, cols])
        if with_counts:
            pltpu.sync_copy(cnt_v, cnt_out.at[wid])

    return agg


_sc_agg_l1 = _make_sc_agg(True)
_sc_agg_l2 = _make_sc_agg(False)

_BLK = 256
_GRID = N_PAD // _BLK


def _scale_rows(inv_lane, s):
    eye = jnp.eye(_BLK, dtype=jnp.float32)
    dn = (((1,), (0,)), ((), ()))
    return lax.dot_general(eye * inv_lane, s, dn,
                           preferred_element_type=jnp.float32)


def _tc_pre1_body(x_ref, wr1_ref, b1_ref, pre_ref):
    dn = (((1,), (1,)), ((), ()))
    pre_ref[...] = lax.dot_general(x_ref[...], wr1_ref[...], dn,
                                   preferred_element_type=jnp.float32) + b1_ref[...]


def _tc_pre2_body(h_ref, wr2_ref, b2_ref, o2_ref):
    dn = (((1,), (1,)), ((), ()))
    o2_ref[...] = lax.dot_general(h_ref[...], wr2_ref[...], dn,
                                  preferred_element_type=jnp.float32) + b2_ref[...]


def _tc1_body(pre_ref, pa_ref, pb_ref, cnt_ref, wl1_ref, wl2_ref,
              h_ref, g2_ref, inv_ref):
    cnt = jnp.sum(cnt_ref[...], axis=0).reshape(1, _BLK)
    inv = 1.0 / jnp.maximum(cnt, 1.0)
    inv_ref[...] = inv.reshape(1, 1, _BLK)
    m = _scale_rows(inv, pa_ref[...] + pb_ref[...])
    dn = (((1,), (1,)), ((), ()))
    pre = (lax.dot_general(m, wl1_ref[...], dn, preferred_element_type=jnp.float32)
           + pre_ref[...])
    h = jnp.maximum(pre, 0.0)
    h_ref[...] = h
    g2_ref[...] = lax.dot_general(h, wl2_ref[...], dn,
                                  preferred_element_type=jnp.float32)


def _tc2_body(qa_ref, qb_ref, inv_ref, o2_ref, o_ref):
    inv = inv_ref[...].reshape(1, _BLK)
    o_ref[...] = _scale_rows(inv, qa_ref[...] + qb_ref[...]) + o2_ref[...]


def _row_spec(d):
    return pl.BlockSpec((_BLK, d), lambda i: (i, 0))


def _full_spec(r, c):
    return pl.BlockSpec((r, c), lambda i: (0, 0))


_CNT_SPEC = pl.BlockSpec((NW, _BLK), lambda i: (0, i))
_INV_SPEC = pl.BlockSpec((1, 1, _BLK), lambda i: (i, 0, 0))

_tc_pre1 = pl.pallas_call(
    _tc_pre1_body,
    grid=(_GRID,),
    in_specs=[_row_spec(D_IN), _full_spec(D_HID, D_IN), _full_spec(1, D_HID)],
    out_specs=_row_spec(D_HID),
    out_shape=jax.ShapeDtypeStruct((N_PAD, D_HID), jnp.float32),
)

_tc_pre2 = pl.pallas_call(
    _tc_pre2_body,
    grid=(_GRID,),
    in_specs=[_row_spec(D_HID), _full_spec(D_OUT, D_HID), _full_spec(1, D_OUT)],
    out_specs=_row_spec(D_OUT),
    out_shape=jax.ShapeDtypeStruct((N_PAD, D_OUT), jnp.float32),
)

_tc1 = pl.pallas_call(
    _tc1_body,
    grid=(_GRID,),
    in_specs=[
        _row_spec(D_HID), _row_spec(D_IN), _row_spec(D_IN),
        _CNT_SPEC,
        _full_spec(D_HID, D_IN),
        _full_spec(D_OUT, D_HID),
    ],
    out_specs=[_row_spec(D_HID), _row_spec(D_OUT), _INV_SPEC],
    out_shape=[
        jax.ShapeDtypeStruct((N_PAD, D_HID), jnp.float32),
        jax.ShapeDtypeStruct((N_PAD, D_OUT), jnp.float32),
        jax.ShapeDtypeStruct((_GRID, 1, _BLK), jnp.float32),
    ],
)

_tc2 = pl.pallas_call(
    _tc2_body,
    grid=(_GRID,),
    in_specs=[
        _row_spec(D_OUT), _row_spec(D_OUT),
        _INV_SPEC,
        _row_spec(D_OUT),
    ],
    out_specs=_row_spec(D_OUT),
    out_shape=jax.ShapeDtypeStruct((N, D_OUT), jnp.float32),
)


def kernel(x, edge_index, Wl1, Wr1, b1, Wl2, Wr2, b2):
    x_pad = jnp.zeros((N_PAD, D_IN), jnp.float32).at[:N].set(x)
    zeros = jnp.zeros((ROWS_PER_TILE, HW), jnp.float32)

    pre1 = _tc_pre1(x_pad, Wr1, b1.reshape(1, D_HID))
    pa, pb, cnt = _sc_agg_l1(x_pad, edge_index, zeros)

    h, g2, inv = _tc1(pre1, pa, pb, cnt, Wl1, Wl2)

    o2 = _tc_pre2(h, Wr2, b2.reshape(1, D_OUT))
    qa, qb = _sc_agg_l2(g2, edge_index, zeros)
    return _tc2(qa, qb, inv, o2)


# speedup vs baseline: 1.9321x; 1.0030x over previous
import functools

import jax
import jax.numpy as jnp
from jax import lax
from jax.experimental import pallas as pl
from jax.experimental.pallas import tpu as pltpu
from jax.experimental.pallas import tpu_sc as plsc

N = 10000
E = 320000
D_IN = 128
D_HID = 256
D_OUT = 128

NC = 2
NS = 16
NW = NC * NS
LANE = 128
HW = 64
N_PAD = 10240
ROWS_PER_TILE = N_PAD // NS
NCH = 80
E_PAD = NW * NCH * LANE
EPT = E_PAD // NW
REAL_LAST = E - (NW - 1) * EPT
NBUF = 2


def _make_sc_agg(with_counts, W=HW, lane=LANE, nch=NCH, nbuf=NBUF):
    npass = D_IN // W
    mesh = plsc.VectorSubcoreMesh(core_axis_name="c", subcore_axis_name="s")
    out_type = [jax.ShapeDtypeStruct((N_PAD, D_IN), jnp.float32)
                for _ in range(NC)]
    if with_counts:
        out_type.append(jax.ShapeDtypeStruct((NW, N_PAD), jnp.float32))
    scratch = [
        pltpu.VMEM((EPT,), jnp.int32),
        pltpu.VMEM((EPT,), jnp.int32),
        [pltpu.VMEM((lane, W), jnp.float32) for _ in range(nbuf)],
        pltpu.VMEM_SHARED((N_PAD, W), jnp.float32),
        pltpu.VMEM_SHARED((N_PAD, W), jnp.float32),
        [pltpu.SemaphoreType.DMA for _ in range(nbuf)],
    ]
    if with_counts:
        scratch.append(pltpu.VMEM((N_PAD,), jnp.float32))

    @functools.partial(
        pl.kernel, out_type=out_type, mesh=mesh, scratch_types=scratch,
        compiler_params=pltpu.CompilerParams(needs_layout_passes=False,
                                             use_tc_tiling_on_sc=False))
    def agg(*args):
        table_hbm, edge, zeros = args[:3]
        rest = args[3:]
        outs = rest[:NC]
        rest = rest[NC:]
        if with_counts:
            cnt_out, src_v, dst_v, bufs, acc, table_sh, gsem, cnt_v = rest
        else:
            src_v, dst_v, bufs, acc, table_sh, gsem = rest
            cnt_out = cnt_v = None
        c = lax.axis_index("c")
        s = lax.axis_index("s")
        wid = s * NC + c
        row0 = s * ROWS_PER_TILE
        rows = pl.ds(row0, ROWS_PER_TILE)
        base = wid * EPT

        @pl.when(wid < NW - 1)
        def _():
            pltpu.sync_copy(edge.at[0, pl.ds(base, EPT)], src_v)
            pltpu.sync_copy(edge.at[1, pl.ds(base, EPT)], dst_v)

        @pl.when(wid == NW - 1)
        def _():
            pltpu.sync_copy(edge.at[0, pl.ds(E - REAL_LAST, REAL_LAST)],
                            src_v.at[pl.ds(0, REAL_LAST)])
            pltpu.sync_copy(edge.at[1, pl.ds(E - REAL_LAST, REAL_LAST)],
                            dst_v.at[pl.ds(0, REAL_LAST)])
            lanes = lax.iota(jnp.int32, 16)

            @pl.loop(0, (EPT - REAL_LAST) // 16)
            def _(i):
                off = REAL_LAST + i * 16
                src_v[pl.ds(off, 16)] = jnp.zeros((16,), jnp.int32)
                dst_v[pl.ds(off, 16)] = N + lax.bitwise_and(off + lanes, 127)

        if with_counts:
            @pl.loop(0, N_PAD // 16)
            def _(i):
                cnt_v[pl.ds(i * 16, 16)] = jnp.zeros((16,), jnp.float32)

        ones16 = jnp.ones((16,), jnp.float32)

        def count(j):
            for g in range(lane // 16):
                d16 = dst_v[pl.ds(j * lane + g * 16, 16)]
                plsc.addupdate_scatter(cnt_v, [d16], ones16)

        for h in range(npass):
            cols = pl.ds(h * W, W)
            do_cnt = with_counts and h == 0
            pltpu.sync_copy(zeros, acc.at[rows])
            pltpu.sync_copy(table_hbm.at[rows, cols], table_sh.at[rows])
            plsc.subcore_barrier()

            def sidx(j):
                return src_v.at[pl.ds(j * lane, lane)]

            def didx(j):
                return dst_v.at[pl.ds(j * lane, lane)]

            for k in range(nbuf - 1):
                pltpu.async_copy(table_sh.at[sidx(k)], bufs[k], gsem[k])

            @pl.loop(0, nch, step=nbuf)
            def _(jj):
                for b in range(nbuf):
                    pf = (b + nbuf - 1) % nbuf

                    @pl.when(jj + b + nbuf - 1 < nch)
                    def _():
                        pltpu.async_copy(table_sh.at[sidx(jj + b + nbuf - 1)],
                                         bufs[pf], gsem[pf])

                    @pl.when(jj + b < nch)
                    def _():
                        pltpu.make_async_copy(table_sh.at[sidx(jj + b)],
                                              bufs[b], gsem[b]).wait()
                        pltpu.sync_copy(bufs[b], acc.at[didx(jj + b)],
                                        add=True)
                        if do_cnt:
                            count(jj + b)

            plsc.subcore_barrier()
            for cc in range(NC):
                @pl.when(c == cc)
                def _():
                    pltpu.sync_copy(acc.at[rows], outs[cc].at[rows, cols])
        if with_counts:
            pltpu.sync_copy(cnt_v, cnt_out.at[wid])

    return agg


_sc_agg_l1 = _make_sc_agg(True)
_sc_agg_l2 = _make_sc_agg(False, nbuf=3)

_BLK = 256
_GRID = N_PAD // _BLK


def _scale_rows(inv_lane, s):
    eye = jnp.eye(_BLK, dtype=jnp.float32)
    dn = (((1,), (0,)), ((), ()))
    return lax.dot_general(eye * inv_lane, s, dn,
                           preferred_element_type=jnp.float32)


def _tc_pre1_body(x_ref, wr1_ref, b1_ref, pre_ref):
    dn = (((1,), (1,)), ((), ()))
    pre_ref[...] = lax.dot_general(x_ref[...], wr1_ref[...], dn,
                                   preferred_element_type=jnp.float32) + b1_ref[...]


def _tc_pre2_body(h_ref, wr2_ref, b2_ref, o2_ref):
    dn = (((1,), (1,)), ((), ()))
    o2_ref[...] = lax.dot_general(h_ref[...], wr2_ref[...], dn,
                                  preferred_element_type=jnp.float32) + b2_ref[...]


def _tc1_body(pre_ref, pa_ref, pb_ref, cnt_ref, wl1_ref, wl2_ref,
              h_ref, g2_ref, inv_ref):
    cnt = jnp.sum(cnt_ref[...], axis=0).reshape(1, _BLK)
    inv = 1.0 / jnp.maximum(cnt, 1.0)
    inv_ref[...] = inv.reshape(1, 1, _BLK)
    m = _scale_rows(inv, pa_ref[...] + pb_ref[...])
    dn = (((1,), (1,)), ((), ()))
    pre = (lax.dot_general(m, wl1_ref[...], dn, preferred_element_type=jnp.float32)
           + pre_ref[...])
    h = jnp.maximum(pre, 0.0)
    h_ref[...] = h
    g2_ref[...] = lax.dot_general(h, wl2_ref[...], dn,
                                  preferred_element_type=jnp.float32)


def _tc2_body(qa_ref, qb_ref, inv_ref, o2_ref, o_ref):
    inv = inv_ref[...].reshape(1, _BLK)
    o_ref[...] = _scale_rows(inv, qa_ref[...] + qb_ref[...]) + o2_ref[...]


def _row_spec(d):
    return pl.BlockSpec((_BLK, d), lambda i: (i, 0))


def _full_spec(r, c):
    return pl.BlockSpec((r, c), lambda i: (0, 0))


_CNT_SPEC = pl.BlockSpec((NW, _BLK), lambda i: (0, i))
_INV_SPEC = pl.BlockSpec((1, 1, _BLK), lambda i: (i, 0, 0))

_tc_pre1 = pl.pallas_call(
    _tc_pre1_body,
    grid=(_GRID,),
    in_specs=[_row_spec(D_IN), _full_spec(D_HID, D_IN), _full_spec(1, D_HID)],
    out_specs=_row_spec(D_HID),
    out_shape=jax.ShapeDtypeStruct((N_PAD, D_HID), jnp.float32),
)

_tc_pre2 = pl.pallas_call(
    _tc_pre2_body,
    grid=(_GRID,),
    in_specs=[_row_spec(D_HID), _full_spec(D_OUT, D_HID), _full_spec(1, D_OUT)],
    out_specs=_row_spec(D_OUT),
    out_shape=jax.ShapeDtypeStruct((N_PAD, D_OUT), jnp.float32),
)

_tc1 = pl.pallas_call(
    _tc1_body,
    grid=(_GRID,),
    in_specs=[
        _row_spec(D_HID), _row_spec(D_IN), _row_spec(D_IN),
        _CNT_SPEC,
        _full_spec(D_HID, D_IN),
        _full_spec(D_OUT, D_HID),
    ],
    out_specs=[_row_spec(D_HID), _row_spec(D_OUT), _INV_SPEC],
    out_shape=[
        jax.ShapeDtypeStruct((N_PAD, D_HID), jnp.float32),
        jax.ShapeDtypeStruct((N_PAD, D_OUT), jnp.float32),
        jax.ShapeDtypeStruct((_GRID, 1, _BLK), jnp.float32),
    ],
)

_tc2 = pl.pallas_call(
    _tc2_body,
    grid=(_GRID,),
    in_specs=[
        _row_spec(D_OUT), _row_spec(D_OUT),
        _INV_SPEC,
        _row_spec(D_OUT),
    ],
    out_specs=_row_spec(D_OUT),
    out_shape=jax.ShapeDtypeStruct((N, D_OUT), jnp.float32),
)


def kernel(x, edge_index, Wl1, Wr1, b1, Wl2, Wr2, b2):
    x_pad = jnp.zeros((N_PAD, D_IN), jnp.float32).at[:N].set(x)
    zeros = jnp.zeros((ROWS_PER_TILE, HW), jnp.float32)

    pre1 = _tc_pre1(x_pad, Wr1, b1.reshape(1, D_HID))
    pa, pb, cnt = _sc_agg_l1(x_pad, edge_index, zeros)

    h, g2, inv = _tc1(pre1, pa, pb, cnt, Wl1, Wl2)

    o2 = _tc_pre2(h, Wr2, b2.reshape(1, D_OUT))
    qa, qb = _sc_agg_l2(g2, edge_index, zeros)
    return _tc2(qa, qb, inv, o2)
